# Initial kernel scaffold; baseline (speedup 1.0000x reference)
import jax
import jax.numpy as jnp
from jax.experimental import pallas as pl


def kernel(features, random_projection, env_indices):
    raise NotImplementedError("write your pallas kernel here")


# speedup vs baseline: 8.8151x; 8.8151x over previous
import functools

import jax
import jax.numpy as jnp
from jax import lax
from jax.experimental import pallas as pl
from jax.experimental.pallas import tpu as pltpu

BATCH = 16384
INPUT_DIM = 128
NUM_BINS = 32

_IBLK = 512
_NBLK = BATCH // _IBLK


def _hash_body(f_ref, p_ref, hash_ref):
    f = f_ref[...]
    n = jnp.float32(BATCH)
    ones = jnp.ones((1, BATCH), jnp.float32)
    s = lax.dot_general(ones, f, (((1,), (0,)), ((), ())),
                        precision=lax.Precision.HIGHEST)
    sq = lax.dot_general(ones, f * f, (((1,), (0,)), ((), ())),
                         precision=lax.Precision.HIGHEST)
    batch_mean = s / n
    batch_var = (sq - s * batch_mean) / (n - 1.0)
    eps_count = jnp.float32(1e-4)
    tot = eps_count + n
    mu = batch_mean * n / tot
    m2 = eps_count + batch_var * n + batch_mean * batch_mean * eps_count * n / tot
    var = m2 / tot
    sigma = jnp.sqrt(var + 1e-8)
    normalized = (f - mu) / sigma
    proj = lax.dot_general(normalized, p_ref[...], (((1,), (0,)), ((), ())),
                           precision=lax.Precision.HIGHEST)
    bits = (proj > 0).astype(jnp.int32)
    k = lax.broadcasted_iota(jnp.int32, (1, NUM_BINS), 1)
    hash_ref[...] = jnp.sum(bits << k, axis=1, keepdims=True, dtype=jnp.int32)


def _count_body(hc_ref, ec_ref, hr_ref, er_ref, out_ref):
    ib = pl.program_id(0)
    hc = hc_ref[...]
    ec = ec_ref[...]

    def jbody(jb, acc):
        hr = hr_ref[0:1, pl.ds(jb * _IBLK, _IBLK)]
        er = er_ref[0:1, pl.ds(jb * _IBLK, _IBLK)]
        eq = (hc == hr) & (ec == er)
        return acc + jnp.sum(eq.astype(jnp.int32), axis=1, keepdims=True,
                             dtype=jnp.int32)

    acc = lax.fori_loop(jnp.int32(0), ib, jbody,
                        jnp.zeros((_IBLK, 1), jnp.int32))
    hr = hr_ref[0:1, pl.ds(ib * _IBLK, _IBLK)]
    er = er_ref[0:1, pl.ds(ib * _IBLK, _IBLK)]
    pos_i = lax.broadcasted_iota(jnp.int32, (_IBLK, _IBLK), 0)
    pos_j = lax.broadcasted_iota(jnp.int32, (_IBLK, _IBLK), 1)
    eq = (hc == hr) & (ec == er) & (pos_j < pos_i)
    acc = acc + jnp.sum(eq.astype(jnp.int32), axis=1, keepdims=True,
                        dtype=jnp.int32)
    occ = (acc + 1).astype(jnp.float32)
    out_ref[...] = lax.rsqrt(occ)


def kernel(features, random_projection, env_indices):
    hash_col = pl.pallas_call(
        _hash_body,
        out_shape=jax.ShapeDtypeStruct((BATCH, 1), jnp.int32),
    )(features.astype(jnp.float32), random_projection.astype(jnp.float32))

    env_col = env_indices.astype(jnp.int32).reshape(BATCH, 1)
    hash_row = hash_col.reshape(1, BATCH)
    env_row = env_col.reshape(1, BATCH)

    rewards = pl.pallas_call(
        _count_body,
        grid=(_NBLK,),
        in_specs=[
            pl.BlockSpec((_IBLK, 1), lambda i: (i, jnp.int32(0))),
            pl.BlockSpec((_IBLK, 1), lambda i: (i, jnp.int32(0))),
            pl.BlockSpec((1, BATCH), lambda i: (jnp.int32(0), jnp.int32(0))),
            pl.BlockSpec((1, BATCH), lambda i: (jnp.int32(0), jnp.int32(0))),
        ],
        out_specs=pl.BlockSpec((_IBLK, 1), lambda i: (i, jnp.int32(0))),
        out_shape=jax.ShapeDtypeStruct((BATCH, 1), jnp.float32),
    )(hash_col, env_col, hash_row, env_row)
    return rewards


# speedup vs baseline: 30.5438x; 3.4649x over previous
import functools

import jax
import jax.numpy as jnp
from jax import lax
from jax.experimental import pallas as pl
from jax.experimental.pallas import tpu as pltpu
from jax.experimental.pallas import tpu_sc as plsc

BATCH = 16384
INPUT_DIM = 128
NUM_BINS = 32

NW = 16
CHUNK = BATCH // NW
FPB = 18
NBINS = 1 << FPB
BINS_PER_W = NBINS // NW
ZCHUNK = 2048
FCAP = CHUNK + 16
NV = CHUNK // 16
RTRASH = NW * CHUNK


def _hash_body(f_ref, p_ref, hash_ref):
    f = f_ref[...]
    n = jnp.float32(BATCH)
    ones = jnp.ones((1, BATCH), jnp.float32)
    s = lax.dot_general(ones, f, (((1,), (0,)), ((), ())),
                        precision=lax.Precision.HIGHEST)
    sq = lax.dot_general(ones, f * f, (((1,), (0,)), ((), ())),
                         precision=lax.Precision.HIGHEST)
    batch_mean = s / n
    batch_var = (sq - s * batch_mean) / (n - 1.0)
    eps_count = jnp.float32(1e-4)
    tot = eps_count + n
    mu = batch_mean * n / tot
    m2 = eps_count + batch_var * n + batch_mean * batch_mean * eps_count * n / tot
    var = m2 / tot
    sigma = jnp.sqrt(var + 1e-8)
    normalized = (f - mu) / sigma
    proj = lax.dot_general(normalized, p_ref[...], (((1,), (0,)), ((), ())),
                           precision=lax.Precision.HIGHEST)
    bits = (proj > 0).astype(jnp.int32)
    k = lax.broadcasted_iota(jnp.int32, (1, NUM_BINS), 1)
    hash_ref[...] = jnp.sum(bits << k, axis=1, keepdims=True, dtype=jnp.int32)


def _i32(x):
    return jnp.int32(x)


def _zeros16():
    return jnp.zeros((16,), jnp.int32)


def _count_sc(hash_hbm, env_hbm, out_hbm,
              h_v, e_v, q_v, bc_v, rw_v, yb_v,
              fp_v, ci_v, ri_v, ones_row, z_v, senti_v, cnt_buf,
              fh_a, fe_a, fq_a, cn_a,
              table, sh_fh, sh_fe, sh_fq, sh_cn, sh_rw):
    wid = lax.axis_index("s")
    base = wid * _i32(CHUNK)
    rowb = wid * _i32(FCAP)

    pltpu.sync_copy(hash_hbm.at[pl.ds(base, CHUNK)], h_v)
    pltpu.sync_copy(env_hbm.at[pl.ds(base, CHUNK)], e_v)

    def or_body(i, c):
        ones_row[pl.ds(i * 16, 16)] = jnp.full((16,), 1, jnp.int32)
        return c
    lax.fori_loop(_i32(0), _i32(8), or_body, _i32(0))

    def z_body(i, c):
        z_v[pl.ds(i * 16, 16)] = _zeros16()
        return c
    lax.fori_loop(_i32(0), _i32(ZCHUNK // 16), z_body, _i32(0))

    def senti_body(i, c):
        senti_v[pl.ds(i * 16, 16)] = jnp.full((16,), -1, jnp.int32)
        return c
    lax.fori_loop(_i32(0), _i32(FCAP // 16), senti_body, _i32(0))

    def fp_body(i, c):
        sl = pl.ds(i * 16, 16)
        h = h_v[sl]
        e = e_v[sl]
        x = h ^ (e * jnp.int32(-1640531527))
        x = x * jnp.int32(-2048144789)
        fp = lax.shift_right_logical(x, jnp.int32(32 - FPB))
        fp_v[i // 8, pl.ds((i % 8) * 16, 16)] = fp
        q_v[sl] = base + i * 16 + lax.iota(jnp.int32, 16)
        rw_v[sl] = jnp.full((16,), 1.0, jnp.float32)
        return c
    lax.fori_loop(_i32(0), _i32(NV), fp_body, _i32(0))

    def zt_body(k, c):
        pltpu.sync_copy(
            z_v, table.at[pl.ds(wid * _i32(BINS_PER_W) + k * _i32(ZCHUNK),
                                ZCHUNK)])
        return c
    lax.fori_loop(_i32(0), _i32(BINS_PER_W // ZCHUNK), zt_body, _i32(0))

    plsc.subcore_barrier()

    def sa_body(k, c):
        pltpu.sync_copy(ones_row, table.at[fp_v.at[k]], add=True)
        return c
    lax.fori_loop(_i32(0), _i32(CHUNK // 128), sa_body, _i32(0))

    plsc.subcore_barrier()

    def ga_body(k, c):
        pltpu.sync_copy(table.at[fp_v.at[k]], bc_v.at[pl.ds(k * 128, 128)])
        return c
    lax.fori_loop(_i32(0), _i32(CHUNK // 128), ga_body, _i32(0))

    pltpu.sync_copy(senti_v, sh_fe.at[pl.ds(rowb, FCAP)])

    lane = lax.iota(jnp.int32, 16)
    trash_c = rowb + _i32(FCAP - 1)

    def c_body(i, cnt):
        sl = pl.ds(i * 16, 16)
        bc = bc_v[sl]
        flag = bc > 1
        fi = jnp.where(flag, jnp.full((16,), 1, jnp.int32), _zeros16())
        excl = _zeros16()
        run = _i32(0)
        for l in range(16):
            excl = jnp.where(lane == _i32(l), _zeros16() + run, excl)
            run = run + fi[l]
        idx = jnp.where(flag, excl + (cnt + rowb), trash_c)
        ci_v[i // 8, pl.ds((i % 8) * 16, 16)] = idx
        return cnt + run
    cnt = lax.fori_loop(_i32(0), _i32(NV), c_body, _i32(0))

    def sc_body(k, c):
        csl = pl.ds(k * 128, 128)
        pltpu.sync_copy(h_v.at[csl], sh_fh.at[ci_v.at[k]])
        pltpu.sync_copy(e_v.at[csl], sh_fe.at[ci_v.at[k]])
        pltpu.sync_copy(q_v.at[csl], sh_fq.at[ci_v.at[k]])
        return c
    lax.fori_loop(_i32(0), _i32(CHUNK // 128), sc_body, _i32(0))

    cnt_buf[...] = _zeros16() + cnt
    pltpu.sync_copy(cnt_buf, sh_cn.at[pl.ds(wid * _i32(16), 16)])
    plsc.subcore_barrier()

    pltpu.sync_copy(sh_fh, fh_a)
    pltpu.sync_copy(sh_fe, fe_a)
    pltpu.sync_copy(sh_fq, fq_a)
    pltpu.sync_copy(sh_cn, cn_a)

    pltpu.sync_copy(rw_v, sh_rw.at[pl.ds(base, CHUNK)])

    def ri_body(i, c):
        ri_v[i // 8, pl.ds((i % 8) * 16, 16)] = _zeros16() + _i32(RTRASH)
        return c
    lax.fori_loop(_i32(0), _i32(NV), ri_body, _i32(0))

    niv = (cnt + _i32(15)) // _i32(16)

    def iv_body(si, c):
        sl = pl.ds(rowb + si * 16, 16)
        ih = fh_a[sl]
        ie = fe_a[sl]
        iq = fq_a[sl]
        ivalid = (si * 16 + lane) < cnt

        def r_body(r, acc):
            cr = cn_a[pl.ds(r * 16, 16)][0]
            njv = (cr + _i32(15)) // _i32(16)
            rb = r * _i32(FCAP)

            def jv_body(jv, acc2):
                jsl = pl.ds(rb + jv * 16, 16)
                jh = fh_a[jsl]
                je = fe_a[jsl]
                jq = fq_a[jsl]
                one16 = jnp.full((16,), 1, jnp.int32)
                for l in range(16):
                    m = (ih == jh[l]) & (ie == je[l]) & (jq[l] < iq)
                    acc2 = acc2 + jnp.where(m, one16, _zeros16())
                return acc2
            return lax.fori_loop(_i32(0), njv, jv_body, acc)

        acc = lax.fori_loop(_i32(0), _i32(NW), r_body, _zeros16())
        occ = (acc + 1).astype(jnp.float32)
        ii = lax.bitcast_convert_type(occ, jnp.int32)
        y = lax.bitcast_convert_type(
            jnp.int32(0x5F3759DF) - lax.shift_right_logical(ii, jnp.int32(1)),
            jnp.float32)
        hx = occ * jnp.float32(0.5)
        y = y * (jnp.float32(1.5) - hx * y * y)
        y = y * (jnp.float32(1.5) - hx * y * y)
        y = y * (jnp.float32(1.5) - hx * y * y)
        gidx = jnp.where(ivalid, iq, _zeros16() + _i32(RTRASH))
        yb_v[pl.ds(si * 16, 16)] = y
        ri_v[si // 8, pl.ds((si % 8) * 16, 16)] = gidx
        return c
    lax.fori_loop(_i32(0), niv, iv_body, _i32(0))

    def rs_body(k, c):
        pltpu.sync_copy(yb_v.at[pl.ds(k * 128, 128)], sh_rw.at[ri_v.at[k]])
        return c
    lax.fori_loop(_i32(0), _i32(CHUNK // 128), rs_body, _i32(0))

    pltpu.sync_copy(sh_rw.at[pl.ds(base, CHUNK)], out_hbm.at[pl.ds(base, CHUNK)])


_count_kernel = functools.partial(
    pl.kernel,
    out_type=jax.ShapeDtypeStruct((BATCH,), jnp.float32),
    mesh=plsc.VectorSubcoreMesh(core_axis_name="c", subcore_axis_name="s",
                                num_cores=1),
    scratch_types=[
        pltpu.VMEM((CHUNK,), jnp.int32),
        pltpu.VMEM((CHUNK,), jnp.int32),
        pltpu.VMEM((CHUNK,), jnp.int32),
        pltpu.VMEM((CHUNK,), jnp.int32),
        pltpu.VMEM((CHUNK,), jnp.float32),
        pltpu.VMEM((CHUNK,), jnp.float32),
        pltpu.VMEM((CHUNK // 128, 128), jnp.int32),
        pltpu.VMEM((CHUNK // 128, 128), jnp.int32),
        pltpu.VMEM((CHUNK // 128, 128), jnp.int32),
        pltpu.VMEM((128,), jnp.int32),
        pltpu.VMEM((ZCHUNK,), jnp.int32),
        pltpu.VMEM((FCAP,), jnp.int32),
        pltpu.VMEM((16,), jnp.int32),
        pltpu.VMEM((NW * FCAP,), jnp.int32),
        pltpu.VMEM((NW * FCAP,), jnp.int32),
        pltpu.VMEM((NW * FCAP,), jnp.int32),
        pltpu.VMEM((NW * 16,), jnp.int32),
        pltpu.VMEM_SHARED((NBINS,), jnp.int32),
        pltpu.VMEM_SHARED((NW * FCAP,), jnp.int32),
        pltpu.VMEM_SHARED((NW * FCAP,), jnp.int32),
        pltpu.VMEM_SHARED((NW * FCAP,), jnp.int32),
        pltpu.VMEM_SHARED((NW * 16,), jnp.int32),
        pltpu.VMEM_SHARED((NW * CHUNK + 16,), jnp.float32),
    ],
)(_count_sc)


def kernel(features, random_projection, env_indices):
    hash_col = pl.pallas_call(
        _hash_body,
        out_shape=jax.ShapeDtypeStruct((BATCH, 1), jnp.int32),
    )(features.astype(jnp.float32), random_projection.astype(jnp.float32))

    hash_flat = hash_col.reshape(BATCH)
    env_flat = env_indices.astype(jnp.int32).reshape(BATCH)
    rewards = _count_kernel(hash_flat, env_flat)
    return rewards.reshape(BATCH, 1)


# speedup vs baseline: 34.5856x; 1.1323x over previous
import functools

import jax
import jax.numpy as jnp
from jax import lax
from jax.experimental import pallas as pl
from jax.experimental.pallas import tpu as pltpu
from jax.experimental.pallas import tpu_sc as plsc

BATCH = 16384
INPUT_DIM = 128
NUM_BINS = 32

NW = 16
CHUNK = BATCH // NW
FPB = 19
NBINS = 1 << FPB
BINS_PER_W = NBINS // NW
ZCHUNK = 2048
FCAP = CHUNK + 16
NV = CHUNK // 16
RTRASH = NW * CHUNK


def _hash_body(f_ref, p_ref, hash_ref):
    f = f_ref[...]
    n = jnp.float32(BATCH)
    ones_col = jnp.ones((BATCH, 1), jnp.float32)
    s = lax.dot_general(f, ones_col, (((0,), (0,)), ((), ())),
                        precision=lax.Precision.HIGHEST)
    sq = lax.dot_general(f * f, ones_col, (((0,), (0,)), ((), ())),
                         precision=lax.Precision.HIGHEST)
    batch_mean = s / n
    batch_var = (sq - s * batch_mean) / (n - 1.0)
    eps_count = jnp.float32(1e-4)
    tot = eps_count + n
    mu = batch_mean * n / tot
    m2 = eps_count + batch_var * n + batch_mean * batch_mean * eps_count * n / tot
    var = m2 / tot
    sigma = jnp.sqrt(var + 1e-8)
    p_scaled = p_ref[...] / sigma
    c = lax.dot_general(p_scaled, mu, (((0,), (0,)), ((), ())),
                        precision=lax.Precision.HIGHEST)
    proj_t = lax.dot_general(p_scaled, f, (((0,), (1,)), ((), ())),
                             precision=lax.Precision.HIGHEST)
    bits = (proj_t > c).astype(jnp.int32)
    k = lax.broadcasted_iota(jnp.int32, (NUM_BINS, 1), 0)
    hash_ref[...] = jnp.sum(bits << k, axis=0, dtype=jnp.int32)


def _i32(x):
    return jnp.int32(x)


def _zeros16():
    return jnp.zeros((16,), jnp.int32)


def _count_sc(hash_hbm, env_hbm, out_hbm,
              h_v, e_v, q_v, bc_v, rw_v, yb_v,
              fp_v, ci_v, ri_v, ones_row, z_v, senti_v, cnt_buf,
              fh_a, fe_a, fq_a, cn_a,
              table, sh_fh, sh_fe, sh_fq, sh_cn, sh_rw):
    wid = lax.axis_index("s")
    base = wid * _i32(CHUNK)
    rowb = wid * _i32(FCAP)

    pltpu.sync_copy(hash_hbm.at[pl.ds(base, CHUNK)], h_v)
    pltpu.sync_copy(env_hbm.at[pl.ds(base, CHUNK)], e_v)

    def or_body(i, c):
        ones_row[pl.ds(i * 16, 16)] = jnp.full((16,), 1, jnp.int32)
        return c
    lax.fori_loop(_i32(0), _i32(8), or_body, _i32(0))

    def z_body(i, c):
        z_v[pl.ds(i * 16, 16)] = _zeros16()
        return c
    lax.fori_loop(_i32(0), _i32(ZCHUNK // 16), z_body, _i32(0))

    def senti_body(i, c):
        senti_v[pl.ds(i * 16, 16)] = jnp.full((16,), -1, jnp.int32)
        return c
    lax.fori_loop(_i32(0), _i32(FCAP // 16), senti_body, _i32(0))

    def fp_body(i, c):
        sl = pl.ds(i * 16, 16)
        h = h_v[sl]
        e = e_v[sl]
        x = h ^ (e * jnp.int32(-1640531527))
        x = x * jnp.int32(-2048144789)
        fp = lax.shift_right_logical(x, jnp.int32(32 - FPB))
        fp_v[i // 8, pl.ds((i % 8) * 16, 16)] = fp
        q_v[sl] = base + i * 16 + lax.iota(jnp.int32, 16)
        rw_v[sl] = jnp.full((16,), 1.0, jnp.float32)
        return c
    lax.fori_loop(_i32(0), _i32(NV), fp_body, _i32(0))

    def zt_body(k, c):
        pltpu.sync_copy(
            z_v, table.at[pl.ds(wid * _i32(BINS_PER_W) + k * _i32(ZCHUNK),
                                ZCHUNK)])
        return c
    lax.fori_loop(_i32(0), _i32(BINS_PER_W // ZCHUNK), zt_body, _i32(0))

    plsc.subcore_barrier()

    def sa_body(k, c):
        pltpu.sync_copy(ones_row, table.at[fp_v.at[k]], add=True)
        return c
    lax.fori_loop(_i32(0), _i32(CHUNK // 128), sa_body, _i32(0))

    plsc.subcore_barrier()

    def ga_body(k, c):
        pltpu.sync_copy(table.at[fp_v.at[k]], bc_v.at[pl.ds(k * 128, 128)])
        return c
    lax.fori_loop(_i32(0), _i32(CHUNK // 128), ga_body, _i32(0))

    pltpu.sync_copy(senti_v, sh_fe.at[pl.ds(rowb, FCAP)])

    lane = lax.iota(jnp.int32, 16)
    trash_c = rowb + _i32(FCAP - 1)

    def c_body(i, cnt):
        sl = pl.ds(i * 16, 16)
        bc = bc_v[sl]
        flag = bc > 1
        fi = jnp.where(flag, jnp.full((16,), 1, jnp.int32), _zeros16())
        excl = _zeros16()
        run = _i32(0)
        for l in range(16):
            excl = jnp.where(lane == _i32(l), _zeros16() + run, excl)
            run = run + fi[l]
        idx = jnp.where(flag, excl + (cnt + rowb), trash_c)
        ci_v[i // 8, pl.ds((i % 8) * 16, 16)] = idx
        return cnt + run
    cnt = lax.fori_loop(_i32(0), _i32(NV), c_body, _i32(0))

    def sc_body(k, c):
        csl = pl.ds(k * 128, 128)
        pltpu.sync_copy(h_v.at[csl], sh_fh.at[ci_v.at[k]])
        pltpu.sync_copy(e_v.at[csl], sh_fe.at[ci_v.at[k]])
        pltpu.sync_copy(q_v.at[csl], sh_fq.at[ci_v.at[k]])
        return c
    lax.fori_loop(_i32(0), _i32(CHUNK // 128), sc_body, _i32(0))

    cnt_buf[...] = _zeros16() + cnt
    pltpu.sync_copy(cnt_buf, sh_cn.at[pl.ds(wid * _i32(16), 16)])
    plsc.subcore_barrier()

    pltpu.sync_copy(sh_fh, fh_a)
    pltpu.sync_copy(sh_fe, fe_a)
    pltpu.sync_copy(sh_fq, fq_a)
    pltpu.sync_copy(sh_cn, cn_a)

    pltpu.sync_copy(rw_v, sh_rw.at[pl.ds(base, CHUNK)])

    def ri_body(i, c):
        ri_v[i // 8, pl.ds((i % 8) * 16, 16)] = _zeros16() + _i32(RTRASH)
        return c
    lax.fori_loop(_i32(0), _i32(NV), ri_body, _i32(0))

    niv = (cnt + _i32(15)) // _i32(16)

    def iv_body(si, c):
        sl = pl.ds(rowb + si * 16, 16)
        ih = fh_a[sl]
        ie = fe_a[sl]
        iq = fq_a[sl]
        ivalid = (si * 16 + lane) < cnt

        def r_body(r, acc):
            cr = cn_a[pl.ds(r * 16, 16)][0]
            njv = (cr + _i32(15)) // _i32(16)
            rb = r * _i32(FCAP)

            def jv_body(jv, acc2):
                jsl = pl.ds(rb + jv * 16, 16)
                jh = fh_a[jsl]
                je = fe_a[jsl]
                jq = fq_a[jsl]
                one16 = jnp.full((16,), 1, jnp.int32)
                for l in range(16):
                    m = (ih == jh[l]) & (ie == je[l]) & (jq[l] < iq)
                    acc2 = acc2 + jnp.where(m, one16, _zeros16())
                return acc2
            return lax.fori_loop(_i32(0), njv, jv_body, acc)

        acc = lax.fori_loop(_i32(0), _i32(NW), r_body, _zeros16())
        occ = (acc + 1).astype(jnp.float32)
        ii = lax.bitcast_convert_type(occ, jnp.int32)
        y = lax.bitcast_convert_type(
            jnp.int32(0x5F3759DF) - lax.shift_right_logical(ii, jnp.int32(1)),
            jnp.float32)
        hx = occ * jnp.float32(0.5)
        y = y * (jnp.float32(1.5) - hx * y * y)
        y = y * (jnp.float32(1.5) - hx * y * y)
        y = y * (jnp.float32(1.5) - hx * y * y)
        gidx = jnp.where(ivalid, iq, _zeros16() + _i32(RTRASH))
        yb_v[pl.ds(si * 16, 16)] = y
        ri_v[si // 8, pl.ds((si % 8) * 16, 16)] = gidx
        return c
    lax.fori_loop(_i32(0), niv, iv_body, _i32(0))

    def rs_body(k, c):
        pltpu.sync_copy(yb_v.at[pl.ds(k * 128, 128)], sh_rw.at[ri_v.at[k]])
        return c
    lax.fori_loop(_i32(0), _i32(CHUNK // 128), rs_body, _i32(0))

    pltpu.sync_copy(sh_rw.at[pl.ds(base, CHUNK)], out_hbm.at[pl.ds(base, CHUNK)])


@functools.cache
def _make_count_kernel():
    return functools.partial(
            pl.kernel,
        out_type=jax.ShapeDtypeStruct((BATCH,), jnp.float32),
        mesh=plsc.VectorSubcoreMesh(core_axis_name="c", subcore_axis_name="s",
                                    num_cores=1),
        scratch_types=[
            pltpu.VMEM((CHUNK,), jnp.int32),
            pltpu.VMEM((CHUNK,), jnp.int32),
            pltpu.VMEM((CHUNK,), jnp.int32),
            pltpu.VMEM((CHUNK,), jnp.int32),
            pltpu.VMEM((CHUNK,), jnp.float32),
            pltpu.VMEM((CHUNK,), jnp.float32),
            pltpu.VMEM((CHUNK // 128, 128), jnp.int32),
            pltpu.VMEM((CHUNK // 128, 128), jnp.int32),
            pltpu.VMEM((CHUNK // 128, 128), jnp.int32),
            pltpu.VMEM((128,), jnp.int32),
            pltpu.VMEM((ZCHUNK,), jnp.int32),
            pltpu.VMEM((FCAP,), jnp.int32),
            pltpu.VMEM((16,), jnp.int32),
            pltpu.VMEM((NW * FCAP,), jnp.int32),
            pltpu.VMEM((NW * FCAP,), jnp.int32),
            pltpu.VMEM((NW * FCAP,), jnp.int32),
            pltpu.VMEM((NW * 16,), jnp.int32),
            pltpu.VMEM_SHARED((NBINS,), jnp.int32),
            pltpu.VMEM_SHARED((NW * FCAP,), jnp.int32),
            pltpu.VMEM_SHARED((NW * FCAP,), jnp.int32),
            pltpu.VMEM_SHARED((NW * FCAP,), jnp.int32),
            pltpu.VMEM_SHARED((NW * 16,), jnp.int32),
            pltpu.VMEM_SHARED((NW * CHUNK + 16,), jnp.float32),
        ],
    )(_count_sc)


def kernel(features, random_projection, env_indices):
    hash_flat = pl.pallas_call(
        _hash_body,
        out_shape=jax.ShapeDtypeStruct((BATCH,), jnp.int32),
    )(features.astype(jnp.float32), random_projection.astype(jnp.float32))

    env_flat = env_indices.astype(jnp.int32).reshape(BATCH)
    rewards = _make_count_kernel()(hash_flat, env_flat)
    return rewards.reshape(BATCH, 1)


# speedup vs baseline: 40.4009x; 1.1681x over previous
import functools

import jax
import jax.numpy as jnp
from jax import lax
from jax.experimental import pallas as pl
from jax.experimental.pallas import tpu as pltpu
from jax.experimental.pallas import tpu_sc as plsc

BATCH = 16384
INPUT_DIM = 128
NUM_BINS = 32

NW = 16
CHUNK = BATCH // NW
FPB = 19
NBINS = 1 << FPB
BINS_PER_W = NBINS // NW
ZCHUNK = 2048
FCAP = CHUNK + 16
NV = CHUNK // 16
RTRASH = NW * CHUNK


def _hash_body(f_ref, p_ref, hash_ref):
    f = f_ref[...]
    n = jnp.float32(BATCH)
    s = jnp.sum(f, axis=0, keepdims=True).reshape(INPUT_DIM, 1)
    sq = jnp.sum(f * f, axis=0, keepdims=True).reshape(INPUT_DIM, 1)
    batch_mean = s / n
    batch_var = (sq - s * batch_mean) / (n - 1.0)
    eps_count = jnp.float32(1e-4)
    tot = eps_count + n
    mu = batch_mean * n / tot
    m2 = eps_count + batch_var * n + batch_mean * batch_mean * eps_count * n / tot
    var = m2 / tot
    sigma = jnp.sqrt(var + 1e-8)
    p_scaled = p_ref[...] / sigma
    c = lax.dot_general(p_scaled, mu, (((0,), (0,)), ((), ())),
                        precision=lax.Precision.HIGHEST)
    proj_t = lax.dot_general(p_scaled, f, (((0,), (1,)), ((), ())),
                             precision=lax.Precision.HIGHEST)
    bits = (proj_t > c).astype(jnp.int32)
    k = lax.broadcasted_iota(jnp.int32, (NUM_BINS, 1), 0)
    hash_ref[...] = jnp.sum(bits << k, axis=0, dtype=jnp.int32)


def _i32(x):
    return jnp.int32(x)


def _zeros16():
    return jnp.zeros((16,), jnp.int32)


def _count_sc(hash_hbm, env_hbm, out_hbm,
              h_v, e_v, q_v, bc_v, rw_v, yb_v,
              fp_v, ci_v, ri_v, ones_row, z_v, senti_v, cnt_buf,
              fh_a, fe_a, fq_a, cn_a,
              table, sh_fh, sh_fe, sh_fq, sh_cn, sh_rw):
    wid = lax.axis_index("s")
    base = wid * _i32(CHUNK)
    rowb = wid * _i32(FCAP)

    pltpu.sync_copy(hash_hbm.at[pl.ds(base, CHUNK)], h_v)
    pltpu.sync_copy(env_hbm.at[pl.ds(base, CHUNK)], e_v)

    def or_body(i, c):
        ones_row[pl.ds(i * 16, 16)] = jnp.full((16,), 1, jnp.int32)
        return c
    lax.fori_loop(_i32(0), _i32(8), or_body, _i32(0))

    def z_body(i, c):
        z_v[pl.ds(i * 16, 16)] = _zeros16()
        return c
    lax.fori_loop(_i32(0), _i32(ZCHUNK // 16), z_body, _i32(0))

    def senti_body(i, c):
        senti_v[pl.ds(i * 16, 16)] = jnp.full((16,), -1, jnp.int32)
        return c
    lax.fori_loop(_i32(0), _i32(FCAP // 16), senti_body, _i32(0))

    def fp_body(i, c):
        sl = pl.ds(i * 16, 16)
        h = h_v[sl]
        e = e_v[sl]
        x = h ^ (e * jnp.int32(-1640531527))
        x = x * jnp.int32(-2048144789)
        fp = lax.shift_right_logical(x, jnp.int32(32 - FPB))
        fp_v[i // 8, pl.ds((i % 8) * 16, 16)] = fp
        q_v[sl] = base + i * 16 + lax.iota(jnp.int32, 16)
        rw_v[sl] = jnp.full((16,), 1.0, jnp.float32)
        return c
    lax.fori_loop(_i32(0), _i32(NV), fp_body, _i32(0))

    def zt_body(k, c):
        pltpu.sync_copy(
            z_v, table.at[pl.ds(wid * _i32(BINS_PER_W) + k * _i32(ZCHUNK),
                                ZCHUNK)])
        return c
    lax.fori_loop(_i32(0), _i32(BINS_PER_W // ZCHUNK), zt_body, _i32(0))

    plsc.subcore_barrier()

    def sa_body(k, c):
        pltpu.sync_copy(ones_row, table.at[fp_v.at[k]], add=True)
        return c
    lax.fori_loop(_i32(0), _i32(CHUNK // 128), sa_body, _i32(0))

    plsc.subcore_barrier()

    def ga_body(k, c):
        pltpu.sync_copy(table.at[fp_v.at[k]], bc_v.at[pl.ds(k * 128, 128)])
        return c
    lax.fori_loop(_i32(0), _i32(CHUNK // 128), ga_body, _i32(0))

    pltpu.sync_copy(senti_v, sh_fe.at[pl.ds(rowb, FCAP)])

    lane = lax.iota(jnp.int32, 16)
    trash_c = rowb + _i32(FCAP - 1)

    def c_body(i, cnt):
        sl = pl.ds(i * 16, 16)
        bc = bc_v[sl]
        flag = bc > 1
        fi = jnp.where(flag, jnp.full((16,), 1, jnp.int32), _zeros16())
        excl = _zeros16()
        run = _i32(0)
        for l in range(16):
            excl = jnp.where(lane == _i32(l), _zeros16() + run, excl)
            run = run + fi[l]
        idx = jnp.where(flag, excl + (cnt + rowb), trash_c)
        ci_v[i // 8, pl.ds((i % 8) * 16, 16)] = idx
        return cnt + run
    cnt = lax.fori_loop(_i32(0), _i32(NV), c_body, _i32(0))

    def sc_body(k, c):
        csl = pl.ds(k * 128, 128)
        pltpu.sync_copy(h_v.at[csl], sh_fh.at[ci_v.at[k]])
        pltpu.sync_copy(e_v.at[csl], sh_fe.at[ci_v.at[k]])
        pltpu.sync_copy(q_v.at[csl], sh_fq.at[ci_v.at[k]])
        return c
    lax.fori_loop(_i32(0), _i32(CHUNK // 128), sc_body, _i32(0))

    cnt_buf[...] = _zeros16() + cnt
    pltpu.sync_copy(cnt_buf, sh_cn.at[pl.ds(wid * _i32(16), 16)])
    plsc.subcore_barrier()

    pltpu.sync_copy(sh_fh, fh_a)
    pltpu.sync_copy(sh_fe, fe_a)
    pltpu.sync_copy(sh_fq, fq_a)
    pltpu.sync_copy(sh_cn, cn_a)

    pltpu.sync_copy(rw_v, sh_rw.at[pl.ds(base, CHUNK)])

    def ri_body(i, c):
        ri_v[i // 8, pl.ds((i % 8) * 16, 16)] = _zeros16() + _i32(RTRASH)
        return c
    lax.fori_loop(_i32(0), _i32(NV), ri_body, _i32(0))

    niv = (cnt + _i32(15)) // _i32(16)

    def iv_body(si, c):
        sl = pl.ds(rowb + si * 16, 16)
        ih = fh_a[sl]
        ie = fe_a[sl]
        iq = fq_a[sl]
        ivalid = (si * 16 + lane) < cnt

        def r_body(r, acc):
            cr = cn_a[pl.ds(r * 16, 16)][0]
            njv = (cr + _i32(15)) // _i32(16)
            rb = r * _i32(FCAP)

            def jv_body(jv, acc2):
                jsl = pl.ds(rb + jv * 16, 16)
                jh = fh_a[jsl]
                je = fe_a[jsl]
                jq = fq_a[jsl]
                one16 = jnp.full((16,), 1, jnp.int32)
                for l in range(16):
                    m = (ih == jh[l]) & (ie == je[l]) & (jq[l] < iq)
                    acc2 = acc2 + jnp.where(m, one16, _zeros16())
                return acc2
            return lax.fori_loop(_i32(0), njv, jv_body, acc)

        acc = lax.fori_loop(_i32(0), _i32(NW), r_body, _zeros16())
        occ = (acc + 1).astype(jnp.float32)
        ii = lax.bitcast_convert_type(occ, jnp.int32)
        y = lax.bitcast_convert_type(
            jnp.int32(0x5F3759DF) - lax.shift_right_logical(ii, jnp.int32(1)),
            jnp.float32)
        hx = occ * jnp.float32(0.5)
        y = y * (jnp.float32(1.5) - hx * y * y)
        y = y * (jnp.float32(1.5) - hx * y * y)
        y = y * (jnp.float32(1.5) - hx * y * y)
        gidx = jnp.where(ivalid, iq, _zeros16() + _i32(RTRASH))
        yb_v[pl.ds(si * 16, 16)] = y
        ri_v[si // 8, pl.ds((si % 8) * 16, 16)] = gidx
        return c
    lax.fori_loop(_i32(0), niv, iv_body, _i32(0))

    def rs_body(k, c):
        pltpu.sync_copy(yb_v.at[pl.ds(k * 128, 128)], sh_rw.at[ri_v.at[k]])
        return c
    lax.fori_loop(_i32(0), _i32(CHUNK // 128), rs_body, _i32(0))

    pltpu.sync_copy(sh_rw.at[pl.ds(base, CHUNK)], out_hbm.at[pl.ds(base, CHUNK)])


@functools.cache
def _make_count_kernel():
    return functools.partial(
            pl.kernel,
        out_type=jax.ShapeDtypeStruct((BATCH,), jnp.float32),
        mesh=plsc.VectorSubcoreMesh(core_axis_name="c", subcore_axis_name="s",
                                    num_cores=1),
        scratch_types=[
            pltpu.VMEM((CHUNK,), jnp.int32),
            pltpu.VMEM((CHUNK,), jnp.int32),
            pltpu.VMEM((CHUNK,), jnp.int32),
            pltpu.VMEM((CHUNK,), jnp.int32),
            pltpu.VMEM((CHUNK,), jnp.float32),
            pltpu.VMEM((CHUNK,), jnp.float32),
            pltpu.VMEM((CHUNK // 128, 128), jnp.int32),
            pltpu.VMEM((CHUNK // 128, 128), jnp.int32),
            pltpu.VMEM((CHUNK // 128, 128), jnp.int32),
            pltpu.VMEM((128,), jnp.int32),
            pltpu.VMEM((ZCHUNK,), jnp.int32),
            pltpu.VMEM((FCAP,), jnp.int32),
            pltpu.VMEM((16,), jnp.int32),
            pltpu.VMEM((NW * FCAP,), jnp.int32),
            pltpu.VMEM((NW * FCAP,), jnp.int32),
            pltpu.VMEM((NW * FCAP,), jnp.int32),
            pltpu.VMEM((NW * 16,), jnp.int32),
            pltpu.VMEM_SHARED((NBINS,), jnp.int32),
            pltpu.VMEM_SHARED((NW * FCAP,), jnp.int32),
            pltpu.VMEM_SHARED((NW * FCAP,), jnp.int32),
            pltpu.VMEM_SHARED((NW * FCAP,), jnp.int32),
            pltpu.VMEM_SHARED((NW * 16,), jnp.int32),
            pltpu.VMEM_SHARED((NW * CHUNK + 16,), jnp.float32),
        ],
    )(_count_sc)


def kernel(features, random_projection, env_indices):
    hash_flat = pl.pallas_call(
        _hash_body,
        out_shape=jax.ShapeDtypeStruct((BATCH,), jnp.int32),
    )(features.astype(jnp.float32), random_projection.astype(jnp.float32))

    env_flat = env_indices.astype(jnp.int32).reshape(BATCH)
    rewards = _make_count_kernel()(hash_flat, env_flat)
    return rewards.reshape(BATCH, 1)


# speedup vs baseline: 43.9398x; 1.0876x over previous
import functools

import jax
import jax.numpy as jnp
from jax import lax
from jax.experimental import pallas as pl
from jax.experimental.pallas import tpu as pltpu
from jax.experimental.pallas import tpu_sc as plsc

BATCH = 16384
INPUT_DIM = 128
NUM_BINS = 32

NW = 16
CHUNK = BATCH // NW
FPB = 19
NBINS = 1 << FPB
BINS_PER_W = NBINS // NW
ZCHUNK = 2048
FCAP = CHUNK + 16
NV = CHUNK // 16
RTRASH = NW * CHUNK


def _hash_body(f_ref, p_ref, hash_ref):
    f = f_ref[...]
    n = jnp.float32(BATCH)
    s = jnp.sum(f, axis=0, keepdims=True).reshape(INPUT_DIM, 1)
    sq = jnp.sum(f * f, axis=0, keepdims=True).reshape(INPUT_DIM, 1)
    batch_mean = s / n
    batch_var = (sq - s * batch_mean) / (n - 1.0)
    eps_count = jnp.float32(1e-4)
    tot = eps_count + n
    mu = batch_mean * n / tot
    m2 = eps_count + batch_var * n + batch_mean * batch_mean * eps_count * n / tot
    var = m2 / tot
    sigma = jnp.sqrt(var + 1e-8)
    p_scaled = p_ref[...] / sigma
    c = lax.dot_general(p_scaled, mu, (((0,), (0,)), ((), ())),
                        precision=lax.Precision.HIGHEST)
    proj_t = lax.dot_general(p_scaled, f, (((0,), (1,)), ((), ())),
                             precision=lax.Precision.HIGHEST)
    bits = (proj_t > c).astype(jnp.int32)
    k = lax.broadcasted_iota(jnp.int32, (NUM_BINS, 1), 0)
    hash_ref[...] = jnp.sum(bits << k, axis=0, dtype=jnp.int32)


def _i32(x):
    return jnp.int32(x)


def _zeros16():
    return jnp.zeros((16,), jnp.int32)


def _count_sc(hash_hbm, env_hbm, out_hbm,
              h_v, e_v, q_v, bc_v, rw_v, yb_v,
              fp_v, ci_v, ri_v, ones_row, z_v, senti_v, cnt_buf,
              fh_a, fe_a, fq_a, cn_a,
              table, sh_fh, sh_fe, sh_fq, sh_cn, sh_rw,
              sem_a, sem_z, sem_s):
    wid = lax.axis_index("s")
    base = wid * _i32(CHUNK)
    rowb = wid * _i32(FCAP)

    dh = pltpu.async_copy(hash_hbm.at[pl.ds(base, CHUNK)], h_v, sem_a)
    de = pltpu.async_copy(env_hbm.at[pl.ds(base, CHUNK)], e_v, sem_a)

    def or_body(i, c):
        ones_row[pl.ds(i * 16, 16)] = jnp.full((16,), 1, jnp.int32)
        return c
    lax.fori_loop(_i32(0), _i32(8), or_body, _i32(0))

    def z_body(i, c):
        z_v[pl.ds(i * 16, 16)] = _zeros16()
        return c
    lax.fori_loop(_i32(0), _i32(ZCHUNK // 16), z_body, _i32(0))

    def senti_body(i, c):
        senti_v[pl.ds(i * 16, 16)] = jnp.full((16,), -1, jnp.int32)
        return c
    lax.fori_loop(_i32(0), _i32(FCAP // 16), senti_body, _i32(0))

    zcs = [
        pltpu.async_copy(
            z_v,
            table.at[pl.ds(wid * _i32(BINS_PER_W) + _i32(k * ZCHUNK), ZCHUNK)],
            sem_z)
        for k in range(BINS_PER_W // ZCHUNK)
    ]

    dh.wait()
    de.wait()

    def fp_body(i, c):
        sl = pl.ds(i * 16, 16)
        h = h_v[sl]
        e = e_v[sl]
        x = h ^ (e * jnp.int32(-1640531527))
        x = x * jnp.int32(-2048144789)
        fp = lax.shift_right_logical(x, jnp.int32(32 - FPB))
        fp_v[i // 8, pl.ds((i % 8) * 16, 16)] = fp
        q_v[sl] = base + i * 16 + lax.iota(jnp.int32, 16)
        rw_v[sl] = jnp.full((16,), 1.0, jnp.float32)
        ri_v[i // 8, pl.ds((i % 8) * 16, 16)] = _zeros16() + _i32(RTRASH)
        return c
    lax.fori_loop(_i32(0), _i32(NV), fp_body, _i32(0))

    sd = pltpu.async_copy(senti_v, sh_fe.at[pl.ds(rowb, FCAP)], sem_a)
    rd = pltpu.async_copy(rw_v, sh_rw.at[pl.ds(base, CHUNK)], sem_a)

    for c in zcs:
        c.wait()
    plsc.subcore_barrier()

    sas = [pltpu.async_copy(ones_row, table.at[fp_v.at[_i32(k)]], sem_s, add=True)
           for k in range(CHUNK // 128)]
    for c in sas:
        c.wait()
    plsc.subcore_barrier()

    gas = [pltpu.async_copy(table.at[fp_v.at[_i32(k)]], bc_v.at[pl.ds(k * 128, 128)],
                            sem_s)
           for k in range(CHUNK // 128)]
    for c in gas:
        c.wait()

    lane = lax.iota(jnp.int32, 16)
    trash_c = rowb + _i32(FCAP - 1)

    def c_body(i, cnt):
        sl = pl.ds(i * 16, 16)
        bc = bc_v[sl]
        flag = bc > 1
        fi = jnp.where(flag, jnp.full((16,), 1, jnp.int32), _zeros16())
        excl = _zeros16()
        run = _i32(0)
        for l in range(16):
            excl = jnp.where(lane == _i32(l), _zeros16() + run, excl)
            run = run + fi[l]
        idx = jnp.where(flag, excl + (cnt + rowb), trash_c)
        ci_v[i // 8, pl.ds((i % 8) * 16, 16)] = idx
        return cnt + run
    cnt = lax.fori_loop(_i32(0), _i32(NV), c_body, _i32(0))

    sd.wait()
    scs = []
    for k in range(CHUNK // 128):
        csl = pl.ds(k * 128, 128)
        scs.append(pltpu.async_copy(h_v.at[csl], sh_fh.at[ci_v.at[_i32(k)]], sem_s))
        scs.append(pltpu.async_copy(e_v.at[csl], sh_fe.at[ci_v.at[_i32(k)]], sem_s))
        scs.append(pltpu.async_copy(q_v.at[csl], sh_fq.at[ci_v.at[_i32(k)]], sem_s))
    cnt_buf[...] = _zeros16() + cnt
    for c in scs:
        c.wait()
    pltpu.sync_copy(cnt_buf, sh_cn.at[pl.ds(wid * _i32(16), 16)])
    plsc.subcore_barrier()

    rbs = [pltpu.async_copy(sh_fh, fh_a, sem_s),
           pltpu.async_copy(sh_fe, fe_a, sem_s),
           pltpu.async_copy(sh_fq, fq_a, sem_s),
           pltpu.async_copy(sh_cn, cn_a, sem_s)]
    for c in rbs:
        c.wait()

    niv = (cnt + _i32(15)) // _i32(16)

    def iv_body(si, c):
        sl = pl.ds(rowb + si * 16, 16)
        ih = fh_a[sl]
        ie = fe_a[sl]
        iq = fq_a[sl]
        ivalid = (si * 16 + lane) < cnt

        def r_body(r, acc):
            cr = cn_a[pl.ds(r * 16, 16)][0]
            njv = (cr + _i32(15)) // _i32(16)
            rb = r * _i32(FCAP)

            def jv_body(jv, acc2):
                jsl = pl.ds(rb + jv * 16, 16)
                jh = fh_a[jsl]
                je = fe_a[jsl]
                jq = fq_a[jsl]
                one16 = jnp.full((16,), 1, jnp.int32)
                for l in range(16):
                    m = (ih == jh[l]) & (ie == je[l]) & (jq[l] < iq)
                    acc2 = acc2 + jnp.where(m, one16, _zeros16())
                return acc2
            return lax.fori_loop(_i32(0), njv, jv_body, acc)

        acc = lax.fori_loop(_i32(0), _i32(NW), r_body, _zeros16())
        occ = (acc + 1).astype(jnp.float32)
        ii = lax.bitcast_convert_type(occ, jnp.int32)
        y = lax.bitcast_convert_type(
            jnp.int32(0x5F3759DF) - lax.shift_right_logical(ii, jnp.int32(1)),
            jnp.float32)
        hx = occ * jnp.float32(0.5)
        y = y * (jnp.float32(1.5) - hx * y * y)
        y = y * (jnp.float32(1.5) - hx * y * y)
        y = y * (jnp.float32(1.5) - hx * y * y)
        gidx = jnp.where(ivalid, iq, _zeros16() + _i32(RTRASH))
        yb_v[pl.ds(si * 16, 16)] = y
        ri_v[si // 8, pl.ds((si % 8) * 16, 16)] = gidx
        return c
    lax.fori_loop(_i32(0), niv, iv_body, _i32(0))

    rd.wait()
    rss = [pltpu.async_copy(yb_v.at[pl.ds(k * 128, 128)],
                            sh_rw.at[ri_v.at[_i32(k)]], sem_s)
           for k in range(CHUNK // 128)]
    for c in rss:
        c.wait()

    pltpu.sync_copy(sh_rw.at[pl.ds(base, CHUNK)], out_hbm.at[pl.ds(base, CHUNK)])


@functools.cache
def _make_count_kernel():
    return functools.partial(
            pl.kernel,
        out_type=jax.ShapeDtypeStruct((BATCH,), jnp.float32),
        mesh=plsc.VectorSubcoreMesh(core_axis_name="c", subcore_axis_name="s",
                                    num_cores=1),
        scratch_types=[
            pltpu.VMEM((CHUNK,), jnp.int32),
            pltpu.VMEM((CHUNK,), jnp.int32),
            pltpu.VMEM((CHUNK,), jnp.int32),
            pltpu.VMEM((CHUNK,), jnp.int32),
            pltpu.VMEM((CHUNK,), jnp.float32),
            pltpu.VMEM((CHUNK,), jnp.float32),
            pltpu.VMEM((CHUNK // 128, 128), jnp.int32),
            pltpu.VMEM((CHUNK // 128, 128), jnp.int32),
            pltpu.VMEM((CHUNK // 128, 128), jnp.int32),
            pltpu.VMEM((128,), jnp.int32),
            pltpu.VMEM((ZCHUNK,), jnp.int32),
            pltpu.VMEM((FCAP,), jnp.int32),
            pltpu.VMEM((16,), jnp.int32),
            pltpu.VMEM((NW * FCAP,), jnp.int32),
            pltpu.VMEM((NW * FCAP,), jnp.int32),
            pltpu.VMEM((NW * FCAP,), jnp.int32),
            pltpu.VMEM((NW * 16,), jnp.int32),
            pltpu.VMEM_SHARED((NBINS,), jnp.int32),
            pltpu.VMEM_SHARED((NW * FCAP,), jnp.int32),
            pltpu.VMEM_SHARED((NW * FCAP,), jnp.int32),
            pltpu.VMEM_SHARED((NW * FCAP,), jnp.int32),
            pltpu.VMEM_SHARED((NW * 16,), jnp.int32),
            pltpu.VMEM_SHARED((NW * CHUNK + 16,), jnp.float32),
            pltpu.SemaphoreType.DMA,
            pltpu.SemaphoreType.DMA,
            pltpu.SemaphoreType.DMA,
        ],
    )(_count_sc)


def kernel(features, random_projection, env_indices):
    hash_flat = pl.pallas_call(
        _hash_body,
        out_shape=jax.ShapeDtypeStruct((BATCH,), jnp.int32),
    )(features.astype(jnp.float32), random_projection.astype(jnp.float32))

    env_flat = env_indices.astype(jnp.int32).reshape(BATCH)
    rewards = _make_count_kernel()(hash_flat, env_flat)
    return rewards.reshape(BATCH, 1)
